# baseline (device time: 309197 ns/iter reference)
import jax
import jax.numpy as jnp
from jax import lax
from jax.experimental import pallas as pl
from jax.experimental.pallas import tpu as pltpu

F_CHUNK = 256
CC_WIDTHS = [1, 1, 2, 4, 8, 8, 4, 2, 1, 1]


def kernel(x, dy):
    m, d = x.shape
    _, f = dy.shape
    dq = d // 4
    n_mm = f // F_CHUNK
    assert sum(CC_WIDTHS) == n_mm
    n_cc = len(CC_WIDTHS)
    cc_off = [sum(CC_WIDTHS[:i]) * F_CHUNK for i in range(n_cc)]
    cc_w = [w * F_CHUNK for w in CC_WIDTHS]
    cc_end = {sum(CC_WIDTHS[: i + 1]) - 1: i for i in range(n_cc)}

    def body(x_ref, dy_ref, out_ref, xs, dyc, p_ref, recv1, recv2,
             xs_sem, load_sems, st1_sems, st2_sems,
             sa_s, sa_r, sb1_s, sb1_r, sb2_s, sb2_r, sb3_s, sb3_r):
        my_x = lax.axis_index("x")
        my_y = lax.axis_index("y")
        my_z = lax.axis_index("z")
        is_owner = my_x == my_y
        r0 = (2 * my_y + my_z) * dq

        x_peer = (1 - my_x, my_y, my_z)
        z_peer = (my_x, my_y, 1 - my_z)
        y_peer = (my_x, 1 - my_y, my_z)

        def cslice(ref, cc):
            return ref.at[:, pl.ds(cc_off[cc], cc_w[cc])]

        def a_rdma(cc):
            return pltpu.make_async_remote_copy(
                src_ref=cslice(p_ref, cc), dst_ref=cslice(recv1, cc),
                send_sem=sa_s.at[cc], recv_sem=sa_r.at[cc],
                device_id=x_peer, device_id_type=pl.DeviceIdType.MESH)

        def b1_rdma(cc):
            return pltpu.make_async_remote_copy(
                src_ref=cslice(p_ref, cc), dst_ref=cslice(recv2, cc),
                send_sem=sb1_s.at[cc], recv_sem=sb1_r.at[cc],
                device_id=z_peer, device_id_type=pl.DeviceIdType.MESH)

        def b2_rdma(cc):
            return pltpu.make_async_remote_copy(
                src_ref=cslice(p_ref, cc), dst_ref=cslice(recv1, cc),
                send_sem=sb2_s.at[cc], recv_sem=sb2_r.at[cc],
                device_id=y_peer, device_id_type=pl.DeviceIdType.MESH)

        def b3_rdma(cc):
            return pltpu.make_async_remote_copy(
                src_ref=cslice(recv1, cc), dst_ref=cslice(recv2, cc),
                send_sem=sb3_s.at[cc], recv_sem=sb3_r.at[cc],
                device_id=z_peer, device_id_type=pl.DeviceIdType.MESH)

        def store1(src, cc):
            pltpu.make_async_copy(
                cslice(src, cc),
                out_ref.at[pl.ds(my_z * dq, dq), pl.ds(cc_off[cc], cc_w[cc])],
                st1_sems.at[cc]).start()

        def store2(cc):
            pltpu.make_async_copy(
                cslice(recv2, cc),
                out_ref.at[pl.ds((1 - my_z) * dq, dq),
                           pl.ds(cc_off[cc], cc_w[cc])],
                st2_sems.at[cc]).start()

        def owner_reduce_and_forward(cc):
            a_rdma(cc).wait_recv()
            sl = pl.ds(cc_off[cc], cc_w[cc])
            p_ref[:, sl] = p_ref[:, sl] + recv1[:, sl]
            b2_rdma(cc).start()
            b1_rdma(cc).start()
            store1(p_ref, cc)

        xcp = pltpu.make_async_copy(x_ref.at[:, pl.ds(r0, dq)], xs, xs_sem)
        xcp.start()

        def dy_load(c):
            return pltpu.make_async_copy(
                dy_ref.at[:, pl.ds(c * F_CHUNK, F_CHUNK)],
                dyc.at[c % 2], load_sems.at[c % 2])

        dy_load(0).start()
        xcp.wait()

        for c in range(n_mm):
            if c + 1 < n_mm:
                dy_load(c + 1).start()
            dy_load(c).wait()
            p_ref[:, pl.ds(c * F_CHUNK, F_CHUNK)] = lax.dot_general(
                xs[:, :], dyc[c % 2, :, :], (((0,), (0,)), ((), ())),
                preferred_element_type=jnp.float32)
            if c in cc_end:
                cc = cc_end[c]

                @pl.when(jnp.logical_not(is_owner))
                def _():
                    a_rdma(cc).start()

                if cc >= 1:
                    @pl.when(is_owner)
                    def _():
                        owner_reduce_and_forward(cc - 1)

        @pl.when(is_owner)
        def _():
            owner_reduce_and_forward(n_cc - 1)
            for cc in range(n_cc):
                b1_rdma(cc).wait_recv()
                store2(cc)
            for cc in range(n_cc):
                b1_rdma(cc).wait_send()
                b2_rdma(cc).wait_send()

        @pl.when(jnp.logical_not(is_owner))
        def _():
            for cc in range(n_cc):
                b2_rdma(cc).wait_recv()
                b3_rdma(cc).start()
                store1(recv1, cc)
            for cc in range(n_cc):
                b3_rdma(cc).wait_recv()
                store2(cc)
            for cc in range(n_cc):
                a_rdma(cc).wait_send()
                b3_rdma(cc).wait_send()

        for cc in range(n_cc):
            pltpu.make_async_copy(
                cslice(recv2, cc),
                out_ref.at[pl.ds(my_z * dq, dq), pl.ds(cc_off[cc], cc_w[cc])],
                st1_sems.at[cc]).wait()
            pltpu.make_async_copy(
                cslice(recv2, cc),
                out_ref.at[pl.ds((1 - my_z) * dq, dq),
                           pl.ds(cc_off[cc], cc_w[cc])],
                st2_sems.at[cc]).wait()

    return pl.pallas_call(
        body,
        out_shape=jax.ShapeDtypeStruct((d // 2, f), jnp.float32),
        in_specs=[
            pl.BlockSpec(memory_space=pl.ANY),
            pl.BlockSpec(memory_space=pl.ANY),
        ],
        out_specs=pl.BlockSpec(memory_space=pl.ANY),
        scratch_shapes=[
            pltpu.VMEM((m, dq), jnp.float32),
            pltpu.VMEM((2, m, F_CHUNK), jnp.float32),
            pltpu.VMEM((dq, f), jnp.float32),
            pltpu.VMEM((dq, f), jnp.float32),
            pltpu.VMEM((dq, f), jnp.float32),
            pltpu.SemaphoreType.DMA,
            pltpu.SemaphoreType.DMA((2,)),
            pltpu.SemaphoreType.DMA((n_cc,)),
            pltpu.SemaphoreType.DMA((n_cc,)),
            pltpu.SemaphoreType.DMA((n_cc,)),
            pltpu.SemaphoreType.DMA((n_cc,)),
            pltpu.SemaphoreType.DMA((n_cc,)),
            pltpu.SemaphoreType.DMA((n_cc,)),
            pltpu.SemaphoreType.DMA((n_cc,)),
            pltpu.SemaphoreType.DMA((n_cc,)),
            pltpu.SemaphoreType.DMA((n_cc,)),
            pltpu.SemaphoreType.DMA((n_cc,)),
        ],
        compiler_params=pltpu.CompilerParams(
            vmem_limit_bytes=60 * 1024 * 1024,
        ),
    )(x, dy)
